# baseline (speedup 1.0000x reference)
import jax
import jax.numpy as jnp
from jax import lax
from jax.experimental import pallas as pl
from jax.experimental.pallas import tpu as pltpu
from jax.experimental.pallas import tpu_sc as plsc

B = 4096
N_DISC, N_CONT = 6, 4
N_FEAT = N_DISC + N_CONT
V1 = 100001
D = 32
NC, NS = 2, 16
NW = NC * NS
BW = B // NW
NK = BW // 16
NG = 3


def _body(*refs):
    idxs = refs[0:3]
    conts = refs[3:6]
    tabs = refs[6:9]
    wbs = refs[9:15]
    outs = refs[15:18]
    idxc = refs[18]
    contc = refs[19]
    idx_feat = refs[20:23]
    gtmp = refs[23:26]
    ctmp = refs[26:29]
    wsc = refs[29:32]
    bsc = refs[32:35]
    gsem = refs[35:38]
    wsem = refs[38]

    wid = lax.axis_index("s") * NC + lax.axis_index("c")
    base = wid * BW
    iot = lax.iota(jnp.int32, 16)
    zeros16 = jnp.zeros((16,), jnp.int32)

    for g in range(NG):
        pltpu.sync_copy(idxs[g].at[pl.ds(base, BW)], idxc)
        pltpu.sync_copy(wbs[2 * g], wsc[g])
        pltpu.sync_copy(wbs[2 * g + 1], bsc[g])
        for i in range(N_DISC):
            cols = jnp.full((16,), i, jnp.int32)
            for k in range(NK):
                rows = k * 16 + iot
                v = plsc.load_gather(idxc, [rows, zeros16, cols]) + 1
                idx_feat[g][i, pl.ds(k * 16, 16)] = v
            pltpu.async_copy(tabs[g].at[i].at[idx_feat[g].at[i]],
                             gtmp[g].at[i], gsem[g])

    for g in range(NG):
        pltpu.sync_copy(conts[g].at[pl.ds(base, BW)], contc)
        wlo = [wsc[g][j, pl.ds(0, 16)] for j in range(N_CONT)]
        whi = [wsc[g][j, pl.ds(16, 16)] for j in range(N_CONT)]
        blo = [bsc[g][j, pl.ds(0, 16)] for j in range(N_CONT)]
        bhi = [bsc[g][j, pl.ds(16, 16)] for j in range(N_CONT)]
        ct = ctmp[g]
        cc = contc

        def cbody(q, carry, ct=ct, cc=cc, wlo=wlo, whi=whi, blo=blo, bhi=bhi):
            cvec = plsc.load_gather(
                cc, [q * 4 + lax.div(iot, 4), zeros16, lax.rem(iot, 4)])
            for rr in range(4):
                r = q * 4 + rr
                for j in range(N_CONT):
                    c = cvec[rr * N_CONT + j]
                    ct[r, j, pl.ds(0, 16)] = c * wlo[j] + blo[j]
                    ct[r, j, pl.ds(16, 16)] = c * whi[j] + bhi[j]
            return carry

        lax.fori_loop(0, BW // 4, cbody, None)

    for g in range(NG):
        for i in range(N_DISC):
            pltpu.make_async_copy(tabs[g].at[i].at[idx_feat[g].at[i]],
                                  gtmp[g].at[i], gsem[g]).wait()
            pltpu.async_copy(gtmp[g].at[i], outs[g].at[pl.ds(base, BW), 0, i],
                             wsem)
        pltpu.async_copy(ctmp[g],
                         outs[g].at[pl.ds(base, BW), 0,
                                    pl.ds(N_DISC, N_CONT)], wsem)
    for g in range(NG):
        for i in range(N_DISC):
            pltpu.make_async_copy(gtmp[g].at[i],
                                  outs[g].at[pl.ds(base, BW), 0, i],
                                  wsem).wait()
        pltpu.make_async_copy(ctmp[g],
                              outs[g].at[pl.ds(base, BW), 0,
                                         pl.ds(N_DISC, N_CONT)], wsem).wait()


@jax.jit
def _impl(qoe_d, ch_d, fu_d, qoe_c, ch_c, fu_c,
          qoe_tab, ch_tab, fu_tab,
          qoe_w, qoe_b, ch_w, ch_b, fu_w, fu_b):
    mesh = plsc.VectorSubcoreMesh(core_axis_name="c", subcore_axis_name="s")
    out_t = [jax.ShapeDtypeStruct((B, 1, N_FEAT, D), jnp.float32)] * 3
    scratch = (
        [pltpu.VMEM((BW, 1, N_DISC), jnp.int32)]
        + [pltpu.VMEM((BW, 1, N_CONT), jnp.float32)]
        + [pltpu.VMEM((N_DISC, BW), jnp.int32)] * 3
        + [pltpu.VMEM((N_DISC, BW, D), jnp.float32)] * 3
        + [pltpu.VMEM((BW, N_CONT, D), jnp.float32)] * 3
        + [pltpu.VMEM((N_CONT, D), jnp.float32)] * 6
        + [pltpu.SemaphoreType.DMA] * 4
    )
    run = pl.kernel(_body, mesh=mesh, out_type=out_t, scratch_types=scratch,
                    compiler_params=pltpu.CompilerParams(
                        use_tc_tiling_on_sc=False,
                        needs_layout_passes=False))
    return tuple(run(qoe_d, ch_d, fu_d, qoe_c, ch_c, fu_c,
                     qoe_tab, ch_tab, fu_tab,
                     qoe_w, qoe_b, ch_w, ch_b, fu_w, fu_b))


def kernel(batch_feature_tensor_target_QOE_discrete,
           batch_feature_tensor_target_CHONGHE_discrete,
           batch_feature_tensor_target_FUFEI_discrete,
           batch_feature_tensor_target_QOE_continue,
           batch_feature_tensor_target_CHONGHE_continue,
           batch_feature_tensor_target_FUFEI_continue,
           qoe_tables, chonghe_tables, fufei_tables,
           qoe_cont_w, qoe_cont_b, chonghe_cont_w, chonghe_cont_b,
           fufei_cont_w, fufei_cont_b):
    return _impl(batch_feature_tensor_target_QOE_discrete,
                 batch_feature_tensor_target_CHONGHE_discrete,
                 batch_feature_tensor_target_FUFEI_discrete,
                 batch_feature_tensor_target_QOE_continue,
                 batch_feature_tensor_target_CHONGHE_continue,
                 batch_feature_tensor_target_FUFEI_continue,
                 qoe_tables, chonghe_tables, fufei_tables,
                 qoe_cont_w, qoe_cont_b, chonghe_cont_w, chonghe_cont_b,
                 fufei_cont_w, fufei_cont_b)


# speedup vs baseline: 4.1053x; 4.1053x over previous
import jax
import jax.numpy as jnp
from jax import lax
from jax.experimental import pallas as pl
from jax.experimental.pallas import tpu as pltpu
from jax.experimental.pallas import tpu_sc as plsc

B = 4096
N_DISC, N_CONT = 6, 4
N_FEAT = N_DISC + N_CONT
V1 = 100001
D = 32
NC, NS = 2, 16
NW = NC * NS
BW = B // NW
NK = BW // 16
NG = 3
BK = 512


def _sc_body(*refs):
    idxs = refs[0:3]
    tabs = refs[3:6]
    outs = refs[6:9]
    istage = refs[9]
    ring = refs[10]
    gslab = refs[11]
    gsems = refs[12]
    wsem = refs[13]

    wid = lax.axis_index("s") * NC + lax.axis_index("c")
    base = pl.multiple_of(wid * BW, BW)

    def fire_chunk(tab, i, k):
        cvec = istage[0, pl.ds(pl.multiple_of(k * 16, 16), 16)]
        half = lax.rem(k, 2)
        for j in range(16):
            s = cvec[j] + 1
            blk = pl.multiple_of((s // 8) * 8, 8)
            slot = pl.multiple_of(half * BW + j * 8, 8)
            pltpu.async_copy(tab.at[i, pl.ds(blk, 8)],
                             ring.at[pl.ds(slot, 8)], gsems.at[half])

    def retire_chunk(tab, il, i, k):
        cvec = istage[0, pl.ds(pl.multiple_of(k * 16, 16), 16)]
        half = lax.rem(k, 2)
        pltpu.make_async_copy(tab.at[i, pl.ds(0, BW)],
                              ring.at[pl.ds(pl.multiple_of(half * BW, 8),
                                            BW)],
                              gsems.at[half]).wait()
        base_r = half * BW
        for j in range(16):
            s = cvec[j] + 1
            rem = lax.rem(s, 8)
            row = base_r + j * 8 + rem
            gslab[il, k * 16 + j, pl.ds(0, 16)] = ring[row, pl.ds(0, 16)]
            gslab[il, k * 16 + j, pl.ds(16, 16)] = ring[row, pl.ds(16, 16)]

    prev_dst = None
    for g in range(NG):
        tab = tabs[g]
        for h in range(2):
            if prev_dst is not None:
                pltpu.make_async_copy(gslab, prev_dst, wsem).wait()

            def iloop(il, carry, g=g, tab=tab, h=h):
                i = h * (N_DISC // 2) + il
                pltpu.sync_copy(idxs[g].at[i, wid], istage)
                fire_chunk(tab, i, 0)

                def kbody(k, c):
                    fire_chunk(tab, i, k)
                    retire_chunk(tab, il, i, k - 1)
                    return c

                lax.fori_loop(1, NK, kbody, None)
                retire_chunk(tab, il, i, NK - 1)
                return carry

            lax.fori_loop(0, N_DISC // 2, iloop, None)
            dst = outs[g].at[pl.ds(h * (N_DISC // 2), N_DISC // 2),
                             pl.ds(base, BW)]
            pltpu.async_copy(gslab, dst, wsem)
            prev_dst = dst

    pltpu.make_async_copy(gslab, prev_dst, wsem).wait()


def _tc_body(d0, d1, d2, c0, c1, c2, w0, b0, w1, b1, w2, b2, o0, o1, o2):
    for d, c, w, bb, o in ((d0, c0, w0, b0, o0), (d1, c1, w1, b1, o1),
                           (d2, c2, w2, b2, o2)):
        for i in range(N_DISC):
            o[:, 0, i, :] = d[i]
        cv = c[:, 0, :]
        for j in range(N_CONT):
            o[:, 0, N_DISC + j, :] = (cv[:, j][:, None] * w[j][None, :]
                                      + bb[j][None, :])


@jax.jit
def _impl(qoe_d, ch_d, fu_d, qoe_c, ch_c, fu_c,
          qoe_tab, ch_tab, fu_tab,
          qoe_w, qoe_b, ch_w, ch_b, fu_w, fu_b):
    idxT = [d.reshape(B, N_DISC).T.reshape(N_DISC, NW, 1, BW)
            for d in (qoe_d, ch_d, fu_d)]

    mesh = plsc.VectorSubcoreMesh(core_axis_name="c", subcore_axis_name="s")
    out_t = [jax.ShapeDtypeStruct((N_DISC, B, D), jnp.float32)] * 3
    scratch = (
        [pltpu.VMEM((1, BW), jnp.int32)]
        + [pltpu.VMEM((2 * BW, D), jnp.float32)]
        + [pltpu.VMEM((N_DISC // 2, BW, D), jnp.float32)]
        + [pltpu.SemaphoreType.DMA((2,)), pltpu.SemaphoreType.DMA]
    )
    run = pl.kernel(_sc_body, mesh=mesh, out_type=out_t,
                    scratch_types=scratch,
                    compiler_params=pltpu.CompilerParams(
                        use_tc_tiling_on_sc=True,
                        needs_layout_passes=False))
    discs = run(idxT[0], idxT[1], idxT[2], qoe_tab, ch_tab, fu_tab)

    grid = (B // BK,)
    dspec = pl.BlockSpec((N_DISC, BK, D), lambda b: (0, b, 0))
    cspec = pl.BlockSpec((BK, 1, N_CONT), lambda b: (b, 0, 0))
    wspec = pl.BlockSpec((N_CONT, D), lambda b: (0, 0))
    ospec = pl.BlockSpec((BK, 1, N_FEAT, D), lambda b: (b, 0, 0, 0))
    outs = pl.pallas_call(
        _tc_body,
        grid=grid,
        in_specs=[dspec] * 3 + [cspec] * 3 + [wspec] * 6,
        out_specs=[ospec] * 3,
        out_shape=[jax.ShapeDtypeStruct((B, 1, N_FEAT, D), jnp.float32)] * 3,
    )(discs[0], discs[1], discs[2], qoe_c, ch_c, fu_c,
      qoe_w, qoe_b, ch_w, ch_b, fu_w, fu_b)
    return tuple(outs)


def kernel(batch_feature_tensor_target_QOE_discrete,
           batch_feature_tensor_target_CHONGHE_discrete,
           batch_feature_tensor_target_FUFEI_discrete,
           batch_feature_tensor_target_QOE_continue,
           batch_feature_tensor_target_CHONGHE_continue,
           batch_feature_tensor_target_FUFEI_continue,
           qoe_tables, chonghe_tables, fufei_tables,
           qoe_cont_w, qoe_cont_b, chonghe_cont_w, chonghe_cont_b,
           fufei_cont_w, fufei_cont_b):
    return _impl(batch_feature_tensor_target_QOE_discrete,
                 batch_feature_tensor_target_CHONGHE_discrete,
                 batch_feature_tensor_target_FUFEI_discrete,
                 batch_feature_tensor_target_QOE_continue,
                 batch_feature_tensor_target_CHONGHE_continue,
                 batch_feature_tensor_target_FUFEI_continue,
                 qoe_tables, chonghe_tables, fufei_tables,
                 qoe_cont_w, qoe_cont_b, chonghe_cont_w, chonghe_cont_b,
                 fufei_cont_w, fufei_cont_b)
